# baseline (speedup 1.0000x reference)
import functools

import jax
import jax.numpy as jnp
from jax import lax
from jax.experimental import pallas as pl
from jax.experimental.pallas import tpu as pltpu
from jax.experimental.pallas import tpu_sc as plsc

B = 1024
T = 200
VOCAB = 100000
N_EMBD = 128
COUNT_DIM = 8
NUM_PROPS = 520
FP_DIM = NUM_PROPS - COUNT_DIM
D_TOT = NUM_PROPS + T

NC, NS = 2, 16
NW = NC * NS
ROWS = B * T
RPW = ROWS // NW
CH = 400
NCH = RPW // CH


def _sc_gather(table, idx):
    mesh = plsc.VectorSubcoreMesh(core_axis_name="c", subcore_axis_name="s")

    @functools.partial(
        pl.kernel,
        mesh=mesh,
        out_type=jax.ShapeDtypeStruct((ROWS, N_EMBD), jnp.float32),
        scratch_types=[
            pltpu.VMEM((RPW,), jnp.int32),
            pltpu.VMEM((CH, N_EMBD), jnp.float32),
            pltpu.VMEM((CH, N_EMBD), jnp.float32),
            pltpu.SemaphoreType.DMA,
            pltpu.SemaphoreType.DMA,
            pltpu.SemaphoreType.DMA,
            pltpu.SemaphoreType.DMA,
        ],
    )
    def k(table_hbm, idx_hbm, out_hbm, idx_v, buf0, buf1, gs0, gs1, ss0, ss1):
        wid = lax.axis_index("s") * NC + lax.axis_index("c")
        base = wid * RPW
        pltpu.sync_copy(idx_hbm.at[pl.ds(base, RPW)], idx_v)
        bufs, gsem, ssem = (buf0, buf1), (gs0, gs1), (ss0, ss1)
        gh = [None] * NCH
        sh = [None] * NCH
        gh[0] = pltpu.async_copy(table_hbm.at[idx_v.at[pl.ds(0, CH)]],
                                 bufs[0], gsem[0])
        for i in range(NCH):
            p = i & 1
            if i + 1 < NCH:
                if i >= 1:
                    sh[i - 1].wait()
                gh[i + 1] = pltpu.async_copy(
                    table_hbm.at[idx_v.at[pl.ds((i + 1) * CH, CH)]],
                    bufs[1 - p], gsem[1 - p])
            gh[i].wait()
            sh[i] = pltpu.async_copy(bufs[p],
                                     out_hbm.at[pl.ds(base + i * CH, CH)],
                                     ssem[p])
        sh[NCH - 2].wait()
        sh[NCH - 1].wait()

    return k(table, idx)


BB = 16


def _prop_body(prop_ref, base_ref, diff_ref, out_ref):
    propf = prop_ref[...].astype(jnp.float32)
    out_ref[...] = (base_ref[...][None]
                    + propf[:, :, None] * diff_ref[...][None])


TB = 40


def _rot_body(g_ref, cos_ref, sin_ref, tt1_ref, o_ref, out_ref):
    del o_ref
    g = g_ref[...]
    h = N_EMBD // 2
    rh = jnp.concatenate([-g[..., h:], g[..., :h]], axis=-1)
    out_ref[...] = (g * cos_ref[...][None] + rh * sin_ref[...][None]
                    + tt1_ref[...][None])


def kernel(token, prop, tok_table, type_table, prop_type_table, cnt_bit,
           cnt_val, fp_pair, fp_bit, fp_val):
    idx = token.reshape(ROWS).astype(jnp.int32)
    gathered = _sc_gather(tok_table, idx).reshape(B, T, N_EMBD)

    inv_freq = 1.0 / (10000.0 ** (jnp.arange(0, N_EMBD, 2, dtype=jnp.float32)
                                  / N_EMBD))
    freqs = jnp.arange(T, dtype=jnp.float32)[:, None] * inv_freq[None, :]
    pos = jnp.concatenate([freqs, freqs], axis=-1)
    cos, sin = jnp.cos(pos), jnp.sin(pos)

    base_cnt = cnt_val[0][None] + cnt_bit + prop_type_table[0][None]
    pair_rep = jnp.repeat(fp_pair, 2, axis=0)
    bit_rep = jnp.tile(fp_bit, (FP_DIM // 2, 1))
    base_fp = fp_val[0][None] + pair_rep + bit_rep + prop_type_table[1][None]
    base = jnp.concatenate([base_cnt, base_fp], axis=0) + type_table[0][None]
    diff = jnp.concatenate([
        jnp.broadcast_to(cnt_val[1] - cnt_val[0], (COUNT_DIM, N_EMBD)),
        jnp.broadcast_to(fp_val[1] - fp_val[0], (FP_DIM, N_EMBD)),
    ], axis=0)
    tt1 = type_table[1][None]

    o1 = pl.pallas_call(
        _prop_body,
        grid=(B // BB,),
        in_specs=[
            pl.BlockSpec((BB, NUM_PROPS), lambda i: (i, 0)),
            pl.BlockSpec((NUM_PROPS, N_EMBD), lambda i: (0, 0)),
            pl.BlockSpec((NUM_PROPS, N_EMBD), lambda i: (0, 0)),
        ],
        out_specs=pl.BlockSpec((BB, NUM_PROPS, N_EMBD), lambda i: (i, 0, 0)),
        out_shape=jax.ShapeDtypeStruct((B, D_TOT, N_EMBD), jnp.float32),
    )(prop, base, diff)

    return pl.pallas_call(
        _rot_body,
        grid=(B // BB, T // TB),
        in_specs=[
            pl.BlockSpec((BB, TB, N_EMBD), lambda i, j: (i, j, 0)),
            pl.BlockSpec((TB, N_EMBD), lambda i, j: (j, 0)),
            pl.BlockSpec((TB, N_EMBD), lambda i, j: (j, 0)),
            pl.BlockSpec((1, N_EMBD), lambda i, j: (0, 0)),
            pl.BlockSpec(memory_space=pl.ANY),
        ],
        out_specs=pl.BlockSpec((BB, TB, N_EMBD),
                               lambda i, j: (i, NUM_PROPS // TB + j, 0)),
        out_shape=jax.ShapeDtypeStruct((B, D_TOT, N_EMBD), jnp.float32),
        input_output_aliases={4: 0},
    )(gathered, cos, sin, tt1, o1)


# speedup vs baseline: 1.7171x; 1.7171x over previous
import functools

import jax
import jax.numpy as jnp
from jax import lax
from jax.experimental import pallas as pl
from jax.experimental.pallas import tpu as pltpu
from jax.experimental.pallas import tpu_sc as plsc

B = 1024
T = 200
VOCAB = 100000
N_EMBD = 128
COUNT_DIM = 8
NUM_PROPS = 520
FP_DIM = NUM_PROPS - COUNT_DIM
D_TOT = NUM_PROPS + T

NC, NS = 2, 16
NW = NC * NS
BPW = B // NW
CB = 1
CH = CB * T
NCH = BPW // CB
L = 16
NCHK = N_EMBD // L
HALF = NCHK // 2


def _rotate_chunk(buf, r, C, S, tt):
    g = [buf[r, pl.ds(L * c, L)] for c in range(NCHK)]
    for c in range(HALF):
        buf[r, pl.ds(L * c, L)] = g[c] * C[c] - g[c + HALF] * S[c] + tt[c]
        buf[r, pl.ds(L * (c + HALF), L)] = (g[c + HALF] * C[c]
                                            + g[c] * S[c] + tt[c + HALF])


def _sc_gather_rotary(table, idx, cosh, sinh, tt1):
    mesh = plsc.VectorSubcoreMesh(core_axis_name="c", subcore_axis_name="s")

    @functools.partial(
        pl.kernel,
        mesh=mesh,
        out_type=jax.ShapeDtypeStruct((B, D_TOT, N_EMBD), jnp.float32),
        scratch_types=[
            pltpu.VMEM((CH,), jnp.int32),
            pltpu.VMEM((CH,), jnp.int32),
            pltpu.VMEM((CH, N_EMBD), jnp.float32),
            pltpu.VMEM((CH, N_EMBD), jnp.float32),
            pltpu.VMEM((T, N_EMBD // 2), jnp.float32),
            pltpu.VMEM((T, N_EMBD // 2), jnp.float32),
            pltpu.VMEM((N_EMBD,), jnp.float32),
            pltpu.SemaphoreType.DMA,
            pltpu.SemaphoreType.DMA,
            pltpu.SemaphoreType.DMA,
            pltpu.SemaphoreType.DMA,
        ],
    )
    def k(table_hbm, idx_hbm, cos_hbm, sin_hbm, tt_hbm, out_hbm,
          idx0, idx1, buf0, buf1, cos_v, sin_v, tt_v, gs0, gs1, ss0, ss1):
        wid = lax.axis_index("s") * NC + lax.axis_index("c")
        b0 = wid * BPW
        pltpu.sync_copy(cos_hbm, cos_v)
        pltpu.sync_copy(sin_hbm, sin_v)
        pltpu.sync_copy(tt_hbm, tt_v)
        tt = [tt_v[pl.ds(L * c, L)] for c in range(NCHK)]
        idxs, bufs = (idx0, idx1), (buf0, buf1)
        gsem, ssem = (gs0, gs1), (ss0, ss1)

        def fetch(i, p):
            pltpu.sync_copy(idx_hbm.at[pl.ds((b0 + i * CB) * T, CH)], idxs[p])
            return pltpu.async_copy(table_hbm.at[idxs[p]], bufs[p], gsem[p])

        gh = [None] * NCH
        sh = [None] * NCH
        gh[0] = fetch(0, 0)
        for i in range(NCH):
            p = i & 1
            if i + 1 < NCH:
                if i >= 1:
                    sh[i - 1].wait()
                gh[i + 1] = fetch(i + 1, 1 - p)
            gh[i].wait()

            def tbody(t, carry, buf=bufs[p]):
                C = [cos_v[t, pl.ds(L * c, L)] for c in range(HALF)]
                S = [sin_v[t, pl.ds(L * c, L)] for c in range(HALF)]
                _rotate_chunk(buf, t, C, S, tt)
                return carry

            lax.fori_loop(0, T, tbody, 0)
            sh[i] = pltpu.async_copy(
                bufs[p], out_hbm.at[b0 + i, pl.ds(NUM_PROPS, T)], ssem[p])
        sh[NCH - 2].wait()
        sh[NCH - 1].wait()

    return k(table, idx, cosh, sinh, tt1)


BB = 16


def _prop_body(prop_ref, base_ref, diff_ref, o_ref, out_ref):
    del o_ref
    propf = prop_ref[...].astype(jnp.float32)
    out_ref[...] = (base_ref[...][None]
                    + propf[:, :, None] * diff_ref[...][None])


def kernel(token, prop, tok_table, type_table, prop_type_table, cnt_bit,
           cnt_val, fp_pair, fp_bit, fp_val):
    idx = token.reshape(B * T).astype(jnp.int32)

    inv_freq = 1.0 / (10000.0 ** (jnp.arange(0, N_EMBD, 2, dtype=jnp.float32)
                                  / N_EMBD))
    freqs = jnp.arange(T, dtype=jnp.float32)[:, None] * inv_freq[None, :]
    cosh, sinh = jnp.cos(freqs), jnp.sin(freqs)
    tt1 = type_table[1]

    o0 = _sc_gather_rotary(tok_table, idx, cosh, sinh, tt1)

    base_cnt = cnt_val[0][None] + cnt_bit + prop_type_table[0][None]
    pair_rep = jnp.repeat(fp_pair, 2, axis=0)
    bit_rep = jnp.tile(fp_bit, (FP_DIM // 2, 1))
    base_fp = fp_val[0][None] + pair_rep + bit_rep + prop_type_table[1][None]
    base = jnp.concatenate([base_cnt, base_fp], axis=0) + type_table[0][None]
    diff = jnp.concatenate([
        jnp.broadcast_to(cnt_val[1] - cnt_val[0], (COUNT_DIM, N_EMBD)),
        jnp.broadcast_to(fp_val[1] - fp_val[0], (FP_DIM, N_EMBD)),
    ], axis=0)

    return pl.pallas_call(
        _prop_body,
        grid=(B // BB,),
        in_specs=[
            pl.BlockSpec((BB, NUM_PROPS), lambda i: (i, 0)),
            pl.BlockSpec((NUM_PROPS, N_EMBD), lambda i: (0, 0)),
            pl.BlockSpec((NUM_PROPS, N_EMBD), lambda i: (0, 0)),
            pl.BlockSpec(memory_space=pl.ANY),
        ],
        out_specs=pl.BlockSpec((BB, NUM_PROPS, N_EMBD), lambda i: (i, 0, 0)),
        out_shape=jax.ShapeDtypeStruct((B, D_TOT, N_EMBD), jnp.float32),
        input_output_aliases={3: 0},
    )(prop, base, diff, o0)
